# baseline (speedup 1.0000x reference)
import functools

import jax
import jax.numpy as jnp
from jax import lax
from jax.experimental import pallas as pl
from jax.experimental.pallas import tpu as pltpu
from jax.experimental.pallas import tpu_sc as plsc

N_FIELDS = 26
VOCAB = 100000
EMBED_DIM = 32
BATCH = 4096

NC, NS, L = 2, 16, 16
NW = NC * NS
B_TOTAL = BATCH * N_FIELDS
B_PER_W = B_TOTAL // NW
PACK = 128 // EMBED_DIM
TAB_ROWS = N_FIELDS * VOCAB // PACK
CHUNK = 416
N_CHUNKS = B_PER_W // CHUNK


def _sc_body(idx_hbm, tab_hbm, out_hbm, idx_v, sub_v, rows_v, sel_v, sem):
    wid = lax.axis_index("s") * NC + lax.axis_index("c")
    base = wid * B_PER_W
    pltpu.sync_copy(idx_hbm.at[pl.ds(base, B_PER_W)], idx_v)

    lane = lax.iota(jnp.int32, L)

    def prep(k, _):
        pos = k * L + lane
        field = lax.rem(pos, N_FIELDS)
        raw = plsc.load_gather(idx_v, [pos])
        plsc.store_scatter(sub_v, [pos], lax.bitwise_and(raw, 3))
        packed = field * (VOCAB // PACK) + lax.shift_right_logical(raw, 2)
        plsc.store_scatter(idx_v, [pos], packed)
        return 0

    lax.fori_loop(0, B_PER_W // L, prep, 0)

    def select_row(i, c0):
        row16 = jnp.full((L,), i, jnp.int32)
        sub16 = plsc.load_gather(sub_v, [c0 + row16])
        col0 = lax.shift_left(sub16, 5) + lane
        for h in range(2):
            val = plsc.load_gather(rows_v, [row16, col0 + h * L])
            plsc.store_scatter(sel_v, [row16, h * L + lane], val)
        return c0

    for c in range(N_CHUNKS):
        cp = pltpu.async_copy(
            tab_hbm.at[idx_v.at[pl.ds(c * CHUNK, CHUNK)]], rows_v, sem)
        cp.wait()
        lax.fori_loop(0, CHUNK, select_row, c * CHUNK)
        pltpu.sync_copy(
            sel_v, out_hbm.at[pl.ds(base + c * CHUNK, CHUNK)])


@jax.jit
def _field_embed(idx1d, tab128):
    run = functools.partial(
        pl.kernel,
        out_type=jax.ShapeDtypeStruct((B_TOTAL, EMBED_DIM), jnp.float32),
        mesh=plsc.VectorSubcoreMesh(core_axis_name="c", subcore_axis_name="s"),
        scratch_types=[
            pltpu.VMEM((B_PER_W,), jnp.int32),
            pltpu.VMEM((B_PER_W,), jnp.int32),
            pltpu.VMEM((CHUNK, 128), jnp.float32),
            pltpu.VMEM((CHUNK, EMBED_DIM), jnp.float32),
            pltpu.SemaphoreType.DMA,
        ],
        compiler_params=pltpu.CompilerParams(needs_layout_passes=False),
    )
    return run(_sc_body)(idx1d, tab128)


def kernel(inputs, tables):
    idx1d = inputs.astype(jnp.int32).reshape(B_TOTAL)
    tab128 = tables.reshape(TAB_ROWS, 128)
    out = _field_embed(idx1d, tab128)
    return out.reshape(BATCH, N_FIELDS, EMBED_DIM)


# speedup vs baseline: 1.2139x; 1.2139x over previous
import functools

import jax
import jax.numpy as jnp
from jax import lax
from jax.experimental import pallas as pl
from jax.experimental.pallas import tpu as pltpu
from jax.experimental.pallas import tpu_sc as plsc

N_FIELDS = 26
VOCAB = 100000
EMBED_DIM = 32
BATCH = 4096

NC, NS, L = 2, 16, 16
NW = NC * NS
N_ROWS = N_FIELDS * EMBED_DIM
N_GROUPS = N_ROWS // 8
TASKS = 4
SUB = 9984
N_SUB = 10
TAIL = VOCAB - N_SUB * SUB
VECS = BATCH // L


def _sc_body(idx_hbm, tab_hbm, out_hbm, idx_spm, idx_f, stage, tail, acc,
             sem):
    cid = lax.axis_index("c")
    sid = lax.axis_index("s")
    wid = sid * NC + cid

    @pl.when(sid == 0)
    def _():
        pltpu.sync_copy(idx_hbm, idx_spm)

    plsc.subcore_barrier()

    lane = lax.iota(jnp.int32, L)

    def make_scan(buf, blen):
        def scan_block(m, v0):
            pos = m * L + lane
            raw = plsc.load_gather(idx_f, [pos])
            loc = raw - v0
            mask = jnp.logical_and(loc >= 0, loc < blen)
            locc = lax.max(lax.min(loc, blen - 1), 0)
            for j in range(8):
                row = jnp.full((L,), j, jnp.int32)
                val = plsc.load_gather(buf, [row, locc], mask=mask)
                plsc.store_scatter(acc, [row, pos], val, mask=mask)
            return v0
        return scan_block

    scan_main = make_scan(stage, SUB)
    scan_tail = make_scan(tail, TAIL)

    for t in range(TASKS):
        g = wid + NW * t

        @pl.when(g < N_GROUPS)
        def _():
            f = g // 4
            e0 = pl.multiple_of((g % 4) * 8, 8)
            r0 = pl.multiple_of(g * 8, 8)
            pltpu.sync_copy(idx_spm.at[f], idx_f)
            for s in range(N_SUB):
                v0 = s * SUB
                pltpu.sync_copy(
                    tab_hbm.at[pl.ds(r0, 8), pl.ds(v0, SUB)], stage)
                lax.fori_loop(0, VECS, scan_main, jnp.int32(v0))
            pltpu.sync_copy(
                tab_hbm.at[pl.ds(r0, 8), pl.ds(N_SUB * SUB, TAIL)], tail)
            lax.fori_loop(0, VECS, scan_tail, jnp.int32(N_SUB * SUB))
            pltpu.sync_copy(acc, out_hbm.at[f, pl.ds(e0, 8)])


@jax.jit
def _field_embed(idx_t, tab_t):
    run = functools.partial(
        pl.kernel,
        out_type=jax.ShapeDtypeStruct((N_FIELDS, EMBED_DIM, BATCH),
                                      jnp.float32),
        mesh=plsc.VectorSubcoreMesh(core_axis_name="c", subcore_axis_name="s"),
        scratch_types=[
            pltpu.VMEM_SHARED((N_FIELDS, BATCH), jnp.int32),
            pltpu.VMEM((BATCH,), jnp.int32),
            pltpu.VMEM((8, SUB), jnp.float32),
            pltpu.VMEM((8, TAIL), jnp.float32),
            pltpu.VMEM((8, BATCH), jnp.float32),
            pltpu.SemaphoreType.DMA,
        ],
        compiler_params=pltpu.CompilerParams(needs_layout_passes=False,
                                             use_tc_tiling_on_sc=False),
    )
    return run(_sc_body)(idx_t, tab_t)


def kernel(inputs, tables):
    idx_t = jnp.transpose(inputs.astype(jnp.int32))
    tab_t = jnp.transpose(tables, (0, 2, 1)).reshape(N_ROWS, VOCAB)
    out_t = _field_embed(idx_t, tab_t)
    return jnp.transpose(out_t, (2, 0, 1))


# speedup vs baseline: 2.1771x; 1.7935x over previous
import functools

import jax
import jax.numpy as jnp
from jax import lax
from jax.experimental import pallas as pl
from jax.experimental.pallas import tpu as pltpu
from jax.experimental.pallas import tpu_sc as plsc

N_FIELDS = 26
VOCAB = 100000
EMBED_DIM = 32
BATCH = 4096

NC, NS, L = 2, 16, 16
NW = NC * NS
N_ROWS = N_FIELDS * EMBED_DIM
N_GROUPS = N_ROWS // 8
SUB = 9984
N_SUB = 10
TAIL = VOCAB - N_SUB * SUB
VECS = BATCH // L
UNROLL = 4


def _sc_body(idx_hbm, tab_hbm, out_hbm, idx_spm, idx_f, stage, tail, acc,
             sem):
    cid = lax.axis_index("c")
    sid = lax.axis_index("s")
    wid = sid * NC + cid

    @pl.when(sid == 0)
    def _():
        pltpu.sync_copy(idx_hbm, idx_spm)

    plsc.subcore_barrier()

    lane = lax.iota(jnp.int32, L)
    rows = [jnp.full((L,), j, jnp.int32) for j in range(8)]

    def make_scan(buf, blen):
        def scan_block(u, v0):
            for k in range(UNROLL):
                pos = (u * UNROLL + k) * L + lane
                raw = plsc.load_gather(idx_f, [pos])
                loc = raw - v0
                mask = jnp.logical_and(loc >= 0, loc < blen)
                locc = lax.max(lax.min(loc, blen - 1), 0)
                for j in range(8):
                    val = plsc.load_gather(buf, [rows[j], locc], mask=mask)
                    plsc.store_scatter(
                        acc,
                        [rows[j], lax.shift_right_logical(pos, 7),
                         lax.bitwise_and(pos, 127)],
                        val, mask=mask)
            return v0
        return scan_block

    scan_main = make_scan(stage, SUB)
    scan_tail = make_scan(tail, TAIL)

    def task(t, _):
        g = wid + NW * t
        f = g // 4
        e0 = pl.multiple_of((g % 4) * 8, 8)
        r0 = pl.multiple_of(g * 8, 8)
        pltpu.sync_copy(idx_spm.at[f], idx_f)
        for s in range(N_SUB):
            v0 = s * SUB
            pltpu.sync_copy(
                tab_hbm.at[pl.ds(r0, 8), pl.ds(v0, SUB)], stage)
            lax.fori_loop(0, VECS // UNROLL, scan_main, jnp.int32(v0))
        pltpu.sync_copy(
            tab_hbm.at[pl.ds(r0, 8), pl.ds(N_SUB * SUB, TAIL)], tail)
        lax.fori_loop(0, VECS // UNROLL, scan_tail, jnp.int32(N_SUB * SUB))
        pltpu.sync_copy(acc, out_hbm.at[f, pl.ds(e0, 8)])
        return 0

    n_tasks = 3 + jnp.where(wid < N_GROUPS - 3 * NW, 1, 0)
    lax.fori_loop(0, n_tasks, task, 0)


@jax.jit
def _field_embed(idx_t, tab_t):
    run = functools.partial(
        pl.kernel,
        out_type=jax.ShapeDtypeStruct(
            (N_FIELDS, EMBED_DIM, BATCH // 128, 128), jnp.float32),
        mesh=plsc.VectorSubcoreMesh(core_axis_name="c", subcore_axis_name="s"),
        scratch_types=[
            pltpu.VMEM_SHARED((32, BATCH), jnp.int32),
            pltpu.VMEM((BATCH,), jnp.int32),
            pltpu.VMEM((8, SUB), jnp.float32),
            pltpu.VMEM((8, TAIL), jnp.float32),
            pltpu.VMEM((8, BATCH // 128, 128), jnp.float32),
            pltpu.SemaphoreType.DMA,
        ],
        compiler_params=pltpu.CompilerParams(needs_layout_passes=False),
    )
    return run(_sc_body)(idx_t, tab_t)


def kernel(inputs, tables):
    idx_t = jnp.pad(jnp.transpose(inputs.astype(jnp.int32)),
                    ((0, 32 - N_FIELDS), (0, 0)))
    tab_t = jnp.transpose(tables, (0, 2, 1)).reshape(N_ROWS, VOCAB)
    out_t = _field_embed(idx_t, tab_t)
    out_t = out_t.reshape(N_FIELDS, EMBED_DIM, BATCH)
    return jnp.transpose(out_t, (2, 0, 1))


# speedup vs baseline: 4.0927x; 1.8799x over previous
import functools

import jax
import jax.numpy as jnp
from jax import lax
from jax.experimental import pallas as pl
from jax.experimental.pallas import tpu as pltpu
from jax.experimental.pallas import tpu_sc as plsc

N_FIELDS = 26
VOCAB = 100000
EMBED_DIM = 32
BATCH = 4096

NC, NS, L = 2, 16, 16
NW = NC * NS
N_ROWS = N_FIELDS * EMBED_DIM
N_GROUPS = N_ROWS // 8
SUB = 9984
N_SUB = 10
TAIL = VOCAB - N_SUB * SUB
VECS = BATCH // L
UNROLL = 4


def _sc_body(idx_hbm, tab_hbm, out_hbm, idx_spm, idx_f, stage, tail, acc,
             sem):
    cid = lax.axis_index("c")
    sid = lax.axis_index("s")
    wid = sid * NC + cid

    @pl.when(sid == 0)
    def _():
        pltpu.sync_copy(idx_hbm, idx_spm)

    plsc.subcore_barrier()

    lane = lax.iota(jnp.int32, L)
    rows = [jnp.full((L,), j, jnp.int32) for j in range(8)]

    def make_scan(buf, blen):
        def scan_block(u, v0):
            poss, masks, loccs = [], [], []
            for k in range(UNROLL):
                pos = (u * UNROLL + k) * L + lane
                raw = plsc.load_gather(idx_f, [pos])
                loc = raw - v0
                poss.append(pos)
                masks.append(jnp.logical_and(loc >= 0, loc < blen))
                loccs.append(lax.max(lax.min(loc, blen - 1), 0))
            vals = [[plsc.load_gather(buf, [rows[j], loccs[k]],
                                      mask=masks[k])
                     for j in range(8)] for k in range(UNROLL)]
            for k in range(UNROLL):
                bt = lax.shift_right_logical(poss[k], 7)
                ln = lax.bitwise_and(poss[k], 127)
                for j in range(8):
                    plsc.store_scatter(acc, [rows[j], bt, ln],
                                       vals[k][j], mask=masks[k])
            return v0
        return scan_block

    scan_main = make_scan(stage, SUB)
    scan_tail = make_scan(tail, TAIL)

    def task(t, _):
        g = wid + NW * t
        f = g // 4
        e0 = pl.multiple_of((g % 4) * 8, 8)
        r0 = pl.multiple_of(g * 8, 8)
        pltpu.sync_copy(idx_spm.at[f], idx_f)
        for s in range(N_SUB):
            v0 = s * SUB
            pltpu.sync_copy(
                tab_hbm.at[pl.ds(r0, 8), pl.ds(v0, SUB)], stage)
            lax.fori_loop(0, VECS // UNROLL, scan_main, jnp.int32(v0))
        pltpu.sync_copy(
            tab_hbm.at[pl.ds(r0, 8), pl.ds(N_SUB * SUB, TAIL)], tail)
        lax.fori_loop(0, VECS // UNROLL, scan_tail, jnp.int32(N_SUB * SUB))
        pltpu.sync_copy(acc, out_hbm.at[f, pl.ds(e0, 8)])
        return 0

    n_tasks = 3 + jnp.where(wid < N_GROUPS - 3 * NW, 1, 0)
    lax.fori_loop(0, n_tasks, task, 0)


@jax.jit
def _field_embed(idx_t, tab_t):
    run = functools.partial(
        pl.kernel,
        out_type=jax.ShapeDtypeStruct(
            (N_FIELDS, EMBED_DIM, BATCH // 128, 128), jnp.float32),
        mesh=plsc.VectorSubcoreMesh(core_axis_name="c", subcore_axis_name="s"),
        scratch_types=[
            pltpu.VMEM_SHARED((32, BATCH), jnp.int32),
            pltpu.VMEM((BATCH,), jnp.int32),
            pltpu.VMEM((8, SUB), jnp.float32),
            pltpu.VMEM((8, TAIL), jnp.float32),
            pltpu.VMEM((8, BATCH // 128, 128), jnp.float32),
            pltpu.SemaphoreType.DMA,
        ],
        compiler_params=pltpu.CompilerParams(needs_layout_passes=False),
    )
    return run(_sc_body)(idx_t, tab_t)


def kernel(inputs, tables):
    idx_t = jnp.pad(jnp.transpose(inputs.astype(jnp.int32)),
                    ((0, 32 - N_FIELDS), (0, 0)))
    tab_t = jnp.transpose(tables, (0, 2, 1)).reshape(N_ROWS, VOCAB)
    out_t = _field_embed(idx_t, tab_t)
    out_t = out_t.reshape(N_FIELDS, EMBED_DIM, BATCH)
    return jnp.transpose(out_t, (2, 0, 1))
